# baseline (speedup 1.0000x reference)
import functools

import jax
import jax.numpy as jnp
from jax import lax
from jax.experimental import pallas as pl
from jax.experimental.pallas import tpu as pltpu
from jax.experimental.pallas import tpu_sc as plsc

_INFO = plsc.get_sparse_core_info()
_NC = _INFO.num_cores
_NS = _INFO.num_subcores
_NW = _NC * _NS


def _make_gather(B, V, D, chunk, nbuf):
    assert B % _NW == 0
    b_per_w = B // _NW
    assert b_per_w % chunk == 0
    n_chunks = b_per_w // chunk
    assert n_chunks >= nbuf

    mesh = plsc.VectorSubcoreMesh(core_axis_name="c", subcore_axis_name="s")

    @functools.partial(
        pl.kernel,
        mesh=mesh,
        out_type=jax.ShapeDtypeStruct((B, D), jnp.float32),
        scratch_types=[
            pltpu.VMEM((b_per_w,), jnp.int32),
            *[pltpu.VMEM((chunk, D), jnp.float32) for _ in range(nbuf)],
            pltpu.SemaphoreType.DMA,
            *[pltpu.SemaphoreType.DMA for _ in range(2 * nbuf)],
        ],
    )
    def gather_kernel(idx_hbm, table_hbm, out_hbm, idx_v, *bufs_and_sems):
        rows = bufs_and_sems[:nbuf]
        sem_idx = bufs_and_sems[nbuf]
        sem_g = bufs_and_sems[nbuf + 1:2 * nbuf + 1]
        sem_s = bufs_and_sems[2 * nbuf + 1:]

        wid = lax.axis_index("s") * _NC + lax.axis_index("c")
        base = wid * b_per_w

        pltpu.async_copy(idx_hbm.at[pl.ds(base, b_per_w)], idx_v,
                         sem_idx).wait()

        def fire_gather(k, buf):
            return pltpu.async_copy(
                table_hbm.at[idx_v.at[pl.ds(k * chunk, chunk)]],
                rows[buf], sem_g[buf])

        def fire_scatter(k, buf):
            return pltpu.async_copy(
                rows[buf], out_hbm.at[pl.ds(base + k * chunk, chunk)],
                sem_s[buf])

        Q = nbuf // 2
        g = [None] * nbuf
        s = [None] * nbuf
        s_pending = [False] * nbuf
        for j in range(Q):
            g[j] = fire_gather(j, j)
        for k in range(n_chunks):
            buf = k % nbuf
            g[buf].wait()
            s[buf] = fire_scatter(k, buf)
            s_pending[buf] = True
            c = k + Q
            if c < n_chunks:
                cb = c % nbuf
                if s_pending[cb]:
                    s[cb].wait()
                    s_pending[cb] = False
                g[cb] = fire_gather(c, cb)
        for j in range(nbuf):
            if s_pending[j]:
                s[j].wait()

    return gather_kernel


def kernel(X, table):
    Bdim, H = X.shape
    V, D = table.shape
    B = Bdim * H
    idx = X.T.reshape(B).astype(jnp.int32)
    out = _make_gather(B, V, D, chunk=200, nbuf=4)(idx, table)
    return out.reshape(H, Bdim, D).transpose(1, 0, 2)


# speedup vs baseline: 1.0137x; 1.0137x over previous
import functools

import jax
import jax.numpy as jnp
from jax import lax
from jax.experimental import pallas as pl
from jax.experimental.pallas import tpu as pltpu
from jax.experimental.pallas import tpu_sc as plsc

_INFO = plsc.get_sparse_core_info()
_NC = _INFO.num_cores
_NS = _INFO.num_subcores
_NW = _NC * _NS


def _make_gather(B, V, D, chunk, nbuf):
    assert B % _NW == 0
    b_per_w = B // _NW
    assert b_per_w % chunk == 0
    n_chunks = b_per_w // chunk
    assert n_chunks >= nbuf

    mesh = plsc.VectorSubcoreMesh(core_axis_name="c", subcore_axis_name="s")

    @functools.partial(
        pl.kernel,
        mesh=mesh,
        compiler_params=pltpu.CompilerParams(skip_device_barrier=True),
        out_type=jax.ShapeDtypeStruct((B, D), jnp.float32),
        scratch_types=[
            pltpu.VMEM((b_per_w,), jnp.int32),
            *[pltpu.VMEM((chunk, D), jnp.float32) for _ in range(nbuf)],
            pltpu.SemaphoreType.DMA,
            *[pltpu.SemaphoreType.DMA for _ in range(2 * nbuf)],
        ],
    )
    def gather_kernel(idx_hbm, table_hbm, out_hbm, idx_v, *bufs_and_sems):
        rows = bufs_and_sems[:nbuf]
        sem_idx = bufs_and_sems[nbuf]
        sem_g = bufs_and_sems[nbuf + 1:2 * nbuf + 1]
        sem_s = bufs_and_sems[2 * nbuf + 1:]

        wid = lax.axis_index("s") * _NC + lax.axis_index("c")
        base = wid * b_per_w

        pltpu.async_copy(idx_hbm.at[pl.ds(base, b_per_w)], idx_v,
                         sem_idx).wait()

        def fire_gather(k, buf):
            return pltpu.async_copy(
                table_hbm.at[idx_v.at[pl.ds(k * chunk, chunk)]],
                rows[buf], sem_g[buf])

        def fire_scatter(k, buf):
            return pltpu.async_copy(
                rows[buf], out_hbm.at[pl.ds(base + k * chunk, chunk)],
                sem_s[buf])

        g = [fire_gather(j, j) for j in range(nbuf)]
        s = [None] * nbuf
        for k in range(n_chunks):
            buf = k % nbuf
            g[buf].wait()
            s[buf] = fire_scatter(k, buf)
            nk = k + nbuf
            if nk < n_chunks:
                s[buf].wait()
                g[buf] = fire_gather(nk, buf)
        for j in range(nbuf):
            buf = (n_chunks - nbuf + j) % nbuf
            s[buf].wait()

    return gather_kernel


def kernel(X, table):
    Bdim, H = X.shape
    V, D = table.shape
    B = Bdim * H
    idx = X.T.reshape(B).astype(jnp.int32)
    out = _make_gather(B, V, D, chunk=400, nbuf=2)(idx, table)
    return out.reshape(H, Bdim, D).transpose(1, 0, 2)


# speedup vs baseline: 1.0140x; 1.0003x over previous
import functools

import jax
import jax.numpy as jnp
from jax import lax
from jax.experimental import pallas as pl
from jax.experimental.pallas import tpu as pltpu
from jax.experimental.pallas import tpu_sc as plsc

_INFO = plsc.get_sparse_core_info()
_NC = _INFO.num_cores
_NS = _INFO.num_subcores
_NW = _NC * _NS


def _make_gather(B, V, D, chunk, nbuf):
    assert B % _NW == 0
    b_per_w = B // _NW
    assert b_per_w % chunk == 0
    n_chunks = b_per_w // chunk
    assert n_chunks >= nbuf

    mesh = plsc.VectorSubcoreMesh(core_axis_name="c", subcore_axis_name="s")

    @functools.partial(
        pl.kernel,
        mesh=mesh,
        out_type=jax.ShapeDtypeStruct((B, D), jnp.float32),
        scratch_types=[
            pltpu.VMEM((b_per_w,), jnp.int32),
            *[pltpu.VMEM((chunk, D), jnp.float32) for _ in range(nbuf)],
            pltpu.SemaphoreType.DMA,
            *[pltpu.SemaphoreType.DMA for _ in range(2 * nbuf)],
        ],
    )
    def gather_kernel(idx_hbm, table_hbm, out_hbm, idx_v, *bufs_and_sems):
        rows = bufs_and_sems[:nbuf]
        sem_idx = bufs_and_sems[nbuf]
        sem_g = bufs_and_sems[nbuf + 1:2 * nbuf + 1]
        sem_s = bufs_and_sems[2 * nbuf + 1:]

        wid = lax.axis_index("s") * _NC + lax.axis_index("c")
        base = wid * b_per_w

        pltpu.async_copy(idx_hbm.at[pl.ds(base, b_per_w)], idx_v,
                         sem_idx).wait()

        def fire_gather(k, buf):
            return pltpu.async_copy(
                table_hbm.at[idx_v.at[pl.ds(k * chunk, chunk)]],
                rows[buf], sem_g[buf])

        def fire_scatter(k, buf):
            return pltpu.async_copy(
                rows[buf], out_hbm.at[pl.ds(base + k * chunk, chunk)],
                sem_s[buf])

        g = [fire_gather(j, j) for j in range(nbuf)]
        s = [None] * nbuf
        for k in range(n_chunks):
            buf = k % nbuf
            g[buf].wait()
            s[buf] = fire_scatter(k, buf)
            nk = k + nbuf
            if nk < n_chunks:
                s[buf].wait()
                g[buf] = fire_gather(nk, buf)
        for j in range(nbuf):
            buf = (n_chunks - nbuf + j) % nbuf
            s[buf].wait()

    return gather_kernel


def kernel(X, table):
    Bdim, H = X.shape
    V, D = table.shape
    B = Bdim * H
    idx = X.T.reshape(B).astype(jnp.int32)
    out = _make_gather(B, V, D, chunk=400, nbuf=2)(idx, table)
    return out.reshape(H, Bdim, D).transpose(1, 0, 2)
